# Initial kernel scaffold; baseline (speedup 1.0000x reference)
import jax
import jax.numpy as jnp
from jax.experimental import pallas as pl


def kernel(ques_h, score, initial_h, W_s, b_s, W_ih, W_hh, b_ih, b_hh):
    raise NotImplementedError("write your pallas kernel here")


# speedup vs baseline: 1.6709x; 1.6709x over previous
import jax
import jax.numpy as jnp
from jax.experimental import pallas as pl
from jax.experimental.pallas import tpu as pltpu

_QUES = 4096
_H = 2048
_R = 512
_JB = _H // _R


def _sp_kernel(sel_ref, q_ref, h_ref, ws_ref, bs_ref, bih_ref, bhh_ref,
               wih_ref, whh_ref, pred_ref, hout_ref, r_scr, z_scr):
    del sel_ref
    gate = pl.program_id(0)
    j = pl.program_id(1)

    gi = jax.lax.dot_general(
        q_ref[...], wih_ref[...], (((1,), (1,)), ((), ())),
        preferred_element_type=jnp.float32) + bih_ref[...]
    gh = jax.lax.dot_general(
        h_ref[...], whh_ref[...], (((1,), (1,)), ((), ())),
        preferred_element_type=jnp.float32) + bhh_ref[...]

    @pl.when(jnp.logical_and(gate == 0, j == 0))
    def _():
        val = (jnp.sum(q_ref[0, :] * ws_ref[0, :_QUES])
               + jnp.sum(h_ref[0, :] * ws_ref[0, _QUES:])
               + bs_ref[0, 0])
        pred_ref[...] = jnp.reshape(val, (1, 1))

    @pl.when(gate == 0)
    def _():
        r_scr[:, pl.ds(j * _R, _R)] = jax.nn.sigmoid(gi + gh)

    @pl.when(gate == 1)
    def _():
        z_scr[:, pl.ds(j * _R, _R)] = jax.nn.sigmoid(gi + gh)

    @pl.when(gate == 2)
    def _():
        r = r_scr[:, pl.ds(j * _R, _R)]
        z = z_scr[:, pl.ds(j * _R, _R)]
        n = jnp.tanh(gi + r * gh)
        h_blk = h_ref[:, pl.ds(j * _R, _R)]
        hout_ref[...] = (1.0 - z) * n + z * h_blk


def kernel(ques_h, score, initial_h, W_s, b_s, W_ih, W_hh, b_ih, b_hh):
    sel = (score < 0.5).astype(jnp.int32)
    q2 = ques_h.reshape(1, _QUES)
    h2 = initial_h.reshape(1, _H)
    bs2 = b_s.reshape(1, 1)
    bih2 = b_ih.reshape(1, 3 * _H)
    bhh2 = b_hh.reshape(1, 3 * _H)

    grid_spec = pltpu.PrefetchScalarGridSpec(
        num_scalar_prefetch=1,
        grid=(3, _JB),
        in_specs=[
            pl.BlockSpec((1, _QUES), lambda i, j, sel: (0, 0)),
            pl.BlockSpec((1, _H), lambda i, j, sel: (0, 0)),
            pl.BlockSpec((1, _QUES + _H), lambda i, j, sel: (0, 0)),
            pl.BlockSpec((1, 1), lambda i, j, sel: (0, 0)),
            pl.BlockSpec((1, _R), lambda i, j, sel: (0, i * _JB + j)),
            pl.BlockSpec((1, _R), lambda i, j, sel: (0, i * _JB + j)),
            pl.BlockSpec((_R, _QUES), lambda i, j, sel: (i * _JB + j, sel[0])),
            pl.BlockSpec((_R, _H), lambda i, j, sel: (i * _JB + j, 0)),
        ],
        out_specs=[
            pl.BlockSpec((1, 1), lambda i, j, sel: (0, 0)),
            pl.BlockSpec((1, _R), lambda i, j, sel: (0, j)),
        ],
        scratch_shapes=[
            pltpu.VMEM((1, _H), jnp.float32),
            pltpu.VMEM((1, _H), jnp.float32),
        ],
    )

    pred2, hout = pl.pallas_call(
        _sp_kernel,
        grid_spec=grid_spec,
        out_shape=[
            jax.ShapeDtypeStruct((1, 1), jnp.float32),
            jax.ShapeDtypeStruct((1, _H), jnp.float32),
        ],
    )(sel, q2, h2, W_s, bs2, bih2, bhh2, W_ih, W_hh)

    return (pred2[0], ques_h, hout.reshape(1, 1, _H))
